# baseline (speedup 1.0000x reference)
import jax
import jax.numpy as jnp
from jax.experimental import pallas as pl

_DEPTH = 1000


def _onehot_body(idx_ref, out_ref):
    idx = idx_ref[...]
    b, s = idx.shape
    iota = jax.lax.broadcasted_iota(jnp.int32, (b, s, _DEPTH), 2)
    out_ref[...] = (iota == idx[:, :, None]).astype(jnp.float32)


def kernel(indices):
    idx32 = indices.astype(jnp.int32)
    n, s = idx32.shape
    block = 64
    grid = n // block
    out = pl.pallas_call(
        _onehot_body,
        grid=(grid,),
        in_specs=[pl.BlockSpec((block, s), lambda i: (i, 0))],
        out_specs=pl.BlockSpec((block, s, _DEPTH), lambda i: (i, 0, 0)),
        out_shape=jax.ShapeDtypeStruct((n, s, _DEPTH), jnp.float32),
    )(idx32)
    return out
